# baseline (speedup 1.0000x reference)
import functools

import jax
import jax.numpy as jnp
from jax import lax
from jax.experimental import pallas as pl
from jax.experimental.pallas import tpu as pltpu
from jax.experimental.pallas import tpu_sc as plsc

N_USERS_K = 5000
N_ITEMS_K = 5000
N_NODES_K = N_USERS_K + N_ITEMS_K
E_K = 320000
D_K = 128
ALPHA_K = 0.5

NC, NS, L = 2, 16, 16
NW = NC * NS
NPT = 320
N_PAD = NW * NPT
CHUNK = 128
NCHUNK = E_K // CHUNK
JG = D_K // L
EROW = 2 * CHUNK
E2 = E_K // 2
NCHUNK2 = E2 // CHUNK
SBC = 64


def _build_ego_body(pref_ref, emb_ref, out_ref):
    out_ref[0:N_USERS_K, :] = pref_ref[...]
    e = emb_ref[...]
    n = jnp.sqrt(jnp.sum(e * e, axis=1, keepdims=True))
    out_ref[N_USERS_K:N_NODES_K, :] = e / jnp.maximum(n, 1e-12)
    out_ref[N_NODES_K:N_PAD, :] = jnp.zeros((N_PAD - N_NODES_K, D_K), jnp.float32)


def _build_ego(pref, emb):
    return pl.pallas_call(
        _build_ego_body,
        out_shape=jax.ShapeDtypeStruct((N_PAD, D_K), jnp.float32),
    )(pref, emb)


_mesh = plsc.VectorSubcoreMesh(core_axis_name="c", subcore_axis_name="s")


@functools.partial(
    pl.kernel,
    out_type=jax.ShapeDtypeStruct((N_PAD, D_K), jnp.float32),
    mesh=_mesh,
    scratch_types=[
        pltpu.VMEM((NPT, D_K), jnp.float32),
        pltpu.VMEM((SBC * EROW,), jnp.int32),
        pltpu.VMEM((SBC * CHUNK,), jnp.float32),
        pltpu.VMEM((CHUNK, D_K), jnp.float32),
        pltpu.VMEM((CHUNK, D_K), jnp.float32),
        pltpu.VMEM((48,), jnp.int32),
        pltpu.SemaphoreType.DMA,
        pltpu.SemaphoreType.DMA,
        pltpu.SemaphoreType.DMA,
    ],
)
def _sc_layer(ego, edata_a, wdata_a, starts_a, edata_b, wdata_b, starts_b,
              out, acc, ebuf, wsbuf, gbuf0, gbuf1, stv, sem_e, sem_g0, sem_g1):
    cid = lax.axis_index("c")
    sid = lax.axis_index("s")
    tid = sid * NC + cid
    base = tid * NPT

    pltpu.sync_copy(ego.at[pl.ds(base, NPT)], acc)

    def _scale_row(r, carry):
        for j in range(JG):
            sl = pl.ds(j * L, L)
            acc[r, sl] = acc[r, sl] * ALPHA_K
        return carry

    lax.fori_loop(0, NPT, _scale_row, 0)

    def _run_list(edata, wdata, starts):
        pltpu.sync_copy(starts, stv)
        start = stv[pl.ds(tid, L)][0]
        end = stv[pl.ds(tid + 1, L)][0]
        c_lo = start // CHUNK
        c_hi = (end + CHUNK - 1) // CHUNK

        def _start_gather(k, gbuf, sem):
            idx = ebuf.at[pl.ds(k * EROW + CHUNK, CHUNK)]
            return pltpu.async_copy(ego.at[idx], gbuf, sem)

        def _compute(k, gbuf):
            def _edge(e, d_carry):
                d = d_carry
                ok = jnp.logical_and(d >= base, d < base + NPT)

                @pl.when(ok)
                def _():
                    w = wsbuf[pl.ds(k * CHUNK + e, L)][0]
                    wv = jnp.full((L,), w, jnp.float32)
                    dl = d - base
                    vals = [gbuf[e, pl.ds(j * L, L)] * wv for j in range(JG)]
                    for j in range(JG):
                        plsc.addupdate(acc.at[dl, pl.ds(j * L, L)], vals[j])

                return ebuf[pl.ds(k * EROW + e + 1, L)][0]

            d0 = ebuf[pl.ds(k * EROW, L)][0]
            lax.fori_loop(0, CHUNK, _edge, d0)

        cs0 = (c_lo // SBC) * SBC

        def _super_block(s_, carry):
            cs = cs0 + s_ * SBC
            k_begin = jnp.maximum(c_lo - cs, 0)
            k_end = jnp.minimum(c_hi - cs, SBC)
            pltpu.async_copy(edata.at[pl.ds(cs * EROW, SBC * EROW)], ebuf,
                             sem_e).wait()
            pltpu.async_copy(wdata.at[pl.ds(cs * CHUNK, SBC * CHUNK)], wsbuf,
                             sem_e).wait()

            @pl.when(k_begin < k_end)
            def _():
                _start_gather(k_begin, gbuf0, sem_g0)

            def _pair(q, pcarry):
                k0 = k_begin + 2 * q
                k1 = k0 + 1

                @pl.when(k1 < k_end)
                def _():
                    _start_gather(k1, gbuf1, sem_g1)

                pltpu.make_async_copy(ego.at[ebuf.at[pl.ds(0, CHUNK)]],
                                      gbuf0, sem_g0).wait()
                _compute(k0, gbuf0)

                @pl.when(k0 + 2 < k_end)
                def _():
                    _start_gather(k0 + 2, gbuf0, sem_g0)

                @pl.when(k1 < k_end)
                def _():
                    pltpu.make_async_copy(
                        ego.at[ebuf.at[pl.ds(0, CHUNK)]],
                        gbuf1, sem_g1).wait()
                    _compute(k1, gbuf1)

                return pcarry

            lax.fori_loop(0, (k_end - k_begin + 1) // 2, _pair, 0)
            return carry

        nsb = (c_hi - cs0 + SBC - 1) // SBC
        lax.fori_loop(0, nsb, _super_block, 0)

    _run_list(edata_a, wdata_a, starts_a)
    _run_list(edata_b, wdata_b, starts_b)

    pltpu.sync_copy(acc, out.at[pl.ds(base, NPT)])


def kernel(edge_index, edge_weight, interaction_preference, interaction_embedding):
    src = edge_index[0].astype(jnp.int32)
    dst = edge_index[1].astype(jnp.int32)
    w = edge_weight[:, 0].astype(jnp.float32)

    def _half(lo):
        d_h = lax.dynamic_slice_in_dim(dst, lo, E2)
        key = (d_h.astype(jnp.uint32) << 18) | jnp.arange(E2, dtype=jnp.uint32)
        key_s = lax.sort(key)
        dst_s = (key_s >> 18).astype(jnp.int32)
        perm = (key_s & jnp.uint32((1 << 18) - 1)).astype(jnp.int32) + lo
        src_s = jnp.take(src, perm)
        w_s = jnp.take(w, perm)
        bounds = jnp.arange(33, dtype=jnp.int32) * NPT
        starts = jnp.searchsorted(dst_s, bounds).astype(jnp.int32)
        starts48 = jnp.concatenate([starts, jnp.full((15,), E2, jnp.int32)])
        edata = jnp.concatenate(
            [dst_s.reshape(NCHUNK2, CHUNK), src_s.reshape(NCHUNK2, CHUNK)],
            axis=1)
        edata = jnp.concatenate(
            [edata, jnp.zeros((SBC, EROW), jnp.int32)], axis=0).reshape(-1)
        wdata = jnp.concatenate([w_s, jnp.zeros((SBC * CHUNK,), jnp.float32)])
        return edata, wdata, starts48

    ea, wa, sa = _half(0)
    eb, wb, sb = _half(E2)

    ego = _build_ego(interaction_preference, interaction_embedding)
    ego = _sc_layer(ego, ea, wa, sa, eb, wb, sb)
    ego = _sc_layer(ego, ea, wa, sa, eb, wb, sb)
    return ego[:N_NODES_K]


# speedup vs baseline: 2.5099x; 2.5099x over previous
import functools

import jax
import jax.numpy as jnp
from jax import lax
from jax.experimental import pallas as pl
from jax.experimental.pallas import tpu as pltpu
from jax.experimental.pallas import tpu_sc as plsc

N_USERS_K = 5000
N_ITEMS_K = 5000
N_NODES_K = N_USERS_K + N_ITEMS_K
E_K = 320000
D_K = 128
ALPHA_K = 0.5

NC, NS, L = 2, 16, 16
NW = NC * NS
NPT = 320
N_PAD = NW * NPT
RPS = N_PAD // NS
CHUNK = 128
NCHUNK = E_K // CHUNK
JG = D_K // L
SBC = 32


def _build_ego_body(pref_ref, emb_ref, out_ref):
    out_ref[0:N_USERS_K, :] = pref_ref[...]
    e = emb_ref[...]
    n = jnp.sqrt(jnp.sum(e * e, axis=1, keepdims=True))
    out_ref[N_USERS_K:N_NODES_K, :] = e / jnp.maximum(n, 1e-12)
    out_ref[N_NODES_K:N_PAD, :] = jnp.zeros((N_PAD - N_NODES_K, D_K), jnp.float32)


def _build_ego(pref, emb):
    return pl.pallas_call(
        _build_ego_body,
        out_shape=jax.ShapeDtypeStruct((N_PAD, D_K), jnp.float32),
    )(pref, emb)


def _merge_body(p_ref, ego_ref, out_ref):
    out_ref[...] = p_ref[0] + p_ref[1] + ALPHA_K * ego_ref[...]


def _merge(partials, ego):
    return pl.pallas_call(
        _merge_body,
        out_shape=jax.ShapeDtypeStruct((N_PAD, D_K), jnp.float32),
    )(partials, ego)


_mesh = plsc.VectorSubcoreMesh(core_axis_name="c", subcore_axis_name="s")


@functools.partial(
    pl.kernel,
    out_type=jax.ShapeDtypeStruct((NC, N_PAD, D_K), jnp.float32),
    mesh=_mesh,
    scratch_types=[
        pltpu.VMEM_SHARED((N_PAD, D_K), jnp.float32),
        pltpu.VMEM((SBC * CHUNK,), jnp.int32),
        pltpu.VMEM((SBC, CHUNK), jnp.int32),
        pltpu.VMEM((SBC * CHUNK,), jnp.float32),
        pltpu.VMEM((CHUNK, D_K), jnp.float32),
        pltpu.VMEM((CHUNK, D_K), jnp.float32),
        pltpu.SemaphoreType.DMA,
        pltpu.SemaphoreType.DMA,
        pltpu.SemaphoreType.DMA,
    ],
)
def _sc_layer(ego, sdata, ddata, wdata, zeros, out, accs, sbuf, dbuf, wsbuf,
              gbuf0, gbuf1, sem_e, sem_g0, sem_g1):
    cid = lax.axis_index("c")
    sid = lax.axis_index("s")
    tid = sid * NC + cid

    pltpu.sync_copy(zeros.at[pl.ds(sid * RPS, RPS)],
                    accs.at[pl.ds(sid * RPS, RPS)])
    plsc.subcore_barrier()

    c_lo = (NCHUNK * tid) // NW
    c_hi = (NCHUNK * (tid + 1)) // NW

    def _start_gather(k, gbuf, sem):
        idx = sbuf.at[pl.ds(k * CHUNK, CHUNK)]
        return pltpu.async_copy(ego.at[idx], gbuf, sem)

    def _compute_scatter(k, gbuf):
        def _edge(e, carry):
            w = wsbuf[pl.ds(k * CHUNK + e, L)][0]
            wv = jnp.full((L,), w, jnp.float32)
            vals = [gbuf[e, pl.ds(j * L, L)] * wv for j in range(JG)]
            for j in range(JG):
                gbuf[e, pl.ds(j * L, L)] = vals[j]
            return carry

        lax.fori_loop(0, CHUNK, _edge, 0)
        pltpu.sync_copy(gbuf, accs.at[dbuf.at[k]], add=True)

    cs0 = (c_lo // SBC) * SBC

    def _super_block(s_, carry):
        cs = cs0 + s_ * SBC
        k_begin = jnp.maximum(c_lo - cs, 0)
        k_end = jnp.minimum(c_hi - cs, SBC)
        pltpu.async_copy(sdata.at[pl.ds(cs * CHUNK, SBC * CHUNK)], sbuf,
                         sem_e).wait()
        pltpu.async_copy(ddata.at[pl.ds(cs, SBC)], dbuf, sem_e).wait()
        pltpu.async_copy(wdata.at[pl.ds(cs * CHUNK, SBC * CHUNK)], wsbuf,
                         sem_e).wait()

        @pl.when(k_begin < k_end)
        def _():
            _start_gather(k_begin, gbuf0, sem_g0)

        def _pair(q, pcarry):
            k0 = k_begin + 2 * q
            k1 = k0 + 1

            @pl.when(k1 < k_end)
            def _():
                _start_gather(k1, gbuf1, sem_g1)

            pltpu.make_async_copy(ego.at[sbuf.at[pl.ds(0, CHUNK)]],
                                  gbuf0, sem_g0).wait()
            _compute_scatter(k0, gbuf0)

            @pl.when(k0 + 2 < k_end)
            def _():
                _start_gather(k0 + 2, gbuf0, sem_g0)

            @pl.when(k1 < k_end)
            def _():
                pltpu.make_async_copy(
                    ego.at[sbuf.at[pl.ds(0, CHUNK)]],
                    gbuf1, sem_g1).wait()
                _compute_scatter(k1, gbuf1)

            return pcarry

        lax.fori_loop(0, (k_end - k_begin + 1) // 2, _pair, 0)
        return carry

    nsb = (c_hi - cs0 + SBC - 1) // SBC
    lax.fori_loop(0, nsb, _super_block, 0)

    plsc.subcore_barrier()
    pltpu.sync_copy(accs.at[pl.ds(sid * RPS, RPS)],
                    out.at[cid, pl.ds(sid * RPS, RPS)])


def kernel(edge_index, edge_weight, interaction_preference, interaction_embedding):
    src = edge_index[0].astype(jnp.int32)
    dst = edge_index[1].astype(jnp.int32)
    w = edge_weight[:, 0].astype(jnp.float32)

    pad = SBC * CHUNK
    sdata = jnp.concatenate([src, jnp.zeros((pad,), jnp.int32)])
    ddata = jnp.concatenate([dst, jnp.zeros((pad,), jnp.int32)]
                            ).reshape(NCHUNK + SBC, CHUNK)
    wdata = jnp.concatenate([w, jnp.zeros((pad,), jnp.float32)])
    zeros = jnp.zeros((N_PAD, D_K), jnp.float32)

    ego = _build_ego(interaction_preference, interaction_embedding)
    parts = _sc_layer(ego, sdata, ddata, wdata, zeros)
    ego = _merge(parts, ego)
    parts = _sc_layer(ego, sdata, ddata, wdata, zeros)
    ego = _merge(parts, ego)
    return ego[:N_NODES_K]
